# baseline (speedup 1.0000x reference)
import functools

import jax
import jax.numpy as jnp
from jax import lax
from jax.experimental import pallas as pl
from jax.experimental.pallas import tpu as pltpu
from jax.experimental.pallas import tpu_sc as plsc

N = 10000
E = 320000
D = 128
NUM_STEPS = 4

NC = 2
NS = 16
NW = NC * NS
CHUNK = 64
CPT = 160
NBUF = 5
E_PAD = NW * CPT * CHUNK
NPAD = 10240
RPT = NPAD // NS

BN = 1000
GRID = N // BN


@functools.partial(
    pl.kernel,
    out_type=jax.ShapeDtypeStruct((NC, NPAD, D), jnp.float32),
    mesh=plsc.VectorSubcoreMesh(core_axis_name="c", subcore_axis_name="s"),
    scratch_types=[
        pltpu.VMEM((NBUF, CHUNK), jnp.int32),
        pltpu.VMEM((NBUF, CHUNK), jnp.int32),
        pltpu.VMEM((NBUF, CHUNK), jnp.int32),
        [pltpu.VMEM((CHUNK, D), jnp.float32) for _ in range(NBUF)],
        pltpu.VMEM_SHARED((NPAD, D), jnp.float32),
        [pltpu.SemaphoreType.DMA for _ in range(NBUF)],
        [pltpu.SemaphoreType.DMA for _ in range(NBUF)],
    ],
)
def _sc_edge_agg(m_hbm, pk_hbm, out_hbm, pk_v, sidx, didx, rows,
                 agg_sh, gsems, isems):
    c = lax.axis_index("c")
    s = lax.axis_index("s")
    wid = c * NS + s

    zv = jnp.zeros((16,), jnp.float32)

    def zrow(i, carry):
        for k in range(D // 16):
            rows[0][i, pl.ds(k * 16, 16)] = zv
        return carry

    lax.fori_loop(0, CHUNK, zrow, 0)

    def zslab(r, carry):
        pltpu.sync_copy(rows[0], agg_sh.at[pl.ds(s * RPT + r * CHUNK, CHUNK)])
        return carry

    lax.fori_loop(0, RPT // CHUNK, zslab, 0)

    plsc.subcore_barrier()

    def fire_idx(j, b):
        pltpu.async_copy(pk_hbm.at[wid, j], pk_v.at[b], isems[b])

    def wait_idx(b):
        pltpu.make_async_copy(pk_hbm.at[wid, 0], pk_v.at[b], isems[b]).wait()

    def unpack(b):
        for k in range(CHUNK // 16):
            w = pk_v[b, pl.ds(k * 16, 16)]
            sidx[b, pl.ds(k * 16, 16)] = w >> 16
            didx[b, pl.ds(k * 16, 16)] = w & 0xFFFF

    def fire(b):
        pltpu.async_copy(m_hbm.at[sidx.at[b]], rows[b], gsems[b])

    def drain(b):
        pltpu.make_async_copy(m_hbm.at[sidx.at[b]], rows[b], gsems[b]).wait()

    def scat(b):
        pltpu.sync_copy(rows[b], agg_sh.at[didx.at[b]], add=True)

    for b in range(NBUF):
        fire_idx(b, b)
    for b in range(NBUF):
        wait_idx(b)
        unpack(b)
        fire(b)

    def body(u, carry):
        for b in range(NBUF):
            j = u * NBUF + b

            @pl.when(j + NBUF < CPT)
            def _():
                fire_idx(j + NBUF, b)
            drain(b)
            scat(b)

            @pl.when(j + NBUF < CPT)
            def _():
                wait_idx(b)
                unpack(b)
                fire(b)
        return carry

    lax.fori_loop(0, CPT // NBUF, body, 0)

    plsc.subcore_barrier()
    pltpu.sync_copy(agg_sh.at[pl.ds(s * RPT, RPT)],
                    out_hbm.at[c, pl.ds(s * RPT, RPT)])


_CONTRACT_LAST = (((1,), (1,)), ((), ()))


def _tc_first(x, w0):
    def body(x_ref, w_ref, m_ref):
        m_ref[...] = jnp.dot(x_ref[...], w_ref[...],
                             preferred_element_type=jnp.float32)

    return pl.pallas_call(
        body,
        grid=(GRID,),
        in_specs=[
            pl.BlockSpec((BN, D), lambda i: (i, 0)),
            pl.BlockSpec((D, D), lambda i: (0, 0)),
        ],
        out_specs=pl.BlockSpec((BN, D), lambda i: (i, 0)),
        out_shape=jax.ShapeDtypeStruct((N, D), jnp.float32),
    )(x, w0)


def _gru_block(agg_ref, h_ref, wih_ref, whh_ref, bih_ref, bhh_ref):
    agg = agg_ref[0] + agg_ref[1]
    h = h_ref[...]
    gi = lax.dot_general(agg, wih_ref[...], _CONTRACT_LAST,
                         preferred_element_type=jnp.float32) + bih_ref[...]
    gh = lax.dot_general(h, whh_ref[...], _CONTRACT_LAST,
                         preferred_element_type=jnp.float32) + bhh_ref[...]
    r = jax.nn.sigmoid(gi[:, 0:D] + gh[:, 0:D])
    z = jax.nn.sigmoid(gi[:, D:2 * D] + gh[:, D:2 * D])
    n = jnp.tanh(gi[:, 2 * D:3 * D] + r * gh[:, 2 * D:3 * D])
    return (1.0 - z) * n + z * h


def _tc_step(agg2, h, w_ih, w_hh, b_ih2, b_hh2, w_next):
    def body(agg_ref, h_ref, wih_ref, whh_ref, bih_ref, bhh_ref, wn_ref,
             hn_ref, mn_ref):
        h_new = _gru_block(agg_ref, h_ref, wih_ref, whh_ref, bih_ref, bhh_ref)
        hn_ref[...] = h_new
        mn_ref[...] = jnp.dot(h_new, wn_ref[...],
                              preferred_element_type=jnp.float32)

    return pl.pallas_call(
        body,
        grid=(GRID,),
        in_specs=[
            pl.BlockSpec((2, BN, D), lambda i: (0, i, 0)),
            pl.BlockSpec((BN, D), lambda i: (i, 0)),
            pl.BlockSpec((3 * D, D), lambda i: (0, 0)),
            pl.BlockSpec((3 * D, D), lambda i: (0, 0)),
            pl.BlockSpec((1, 3 * D), lambda i: (0, 0)),
            pl.BlockSpec((1, 3 * D), lambda i: (0, 0)),
            pl.BlockSpec((D, D), lambda i: (0, 0)),
        ],
        out_specs=[
            pl.BlockSpec((BN, D), lambda i: (i, 0)),
            pl.BlockSpec((BN, D), lambda i: (i, 0)),
        ],
        out_shape=[
            jax.ShapeDtypeStruct((N, D), jnp.float32),
            jax.ShapeDtypeStruct((N, D), jnp.float32),
        ],
    )(agg2, h, w_ih, w_hh, b_ih2, b_hh2, w_next)


def _tc_last(agg2, h, w_ih, w_hh, b_ih2, b_hh2, cls_w_pad, cls_b_pad):
    def body(agg_ref, h_ref, wih_ref, whh_ref, bih_ref, bhh_ref, cw_ref,
             cb_ref, pool_ref, probs_ref):
        i = pl.program_id(0)
        h_new = _gru_block(agg_ref, h_ref, wih_ref, whh_ref, bih_ref, bhh_ref)
        blk_max = jnp.max(jax.nn.relu(h_new), axis=0, keepdims=True)

        @pl.when(i == 0)
        def _():
            pool_ref[...] = blk_max

        @pl.when(i > 0)
        def _():
            pool_ref[...] = jnp.maximum(pool_ref[...], blk_max)

        @pl.when(i == GRID - 1)
        def _():
            logits = lax.dot_general(pool_ref[...], cw_ref[...],
                                     _CONTRACT_LAST,
                                     preferred_element_type=jnp.float32)
            logits = logits + cb_ref[...]
            e = jnp.exp(logits - jnp.max(logits, axis=1, keepdims=True))
            probs_ref[...] = e / jnp.sum(e, axis=1, keepdims=True)

    _, probs = pl.pallas_call(
        body,
        grid=(GRID,),
        in_specs=[
            pl.BlockSpec((2, BN, D), lambda i: (0, i, 0)),
            pl.BlockSpec((BN, D), lambda i: (i, 0)),
            pl.BlockSpec((3 * D, D), lambda i: (0, 0)),
            pl.BlockSpec((3 * D, D), lambda i: (0, 0)),
            pl.BlockSpec((1, 3 * D), lambda i: (0, 0)),
            pl.BlockSpec((1, 3 * D), lambda i: (0, 0)),
            pl.BlockSpec((D, D), lambda i: (0, 0)),
            pl.BlockSpec((1, D), lambda i: (0, 0)),
        ],
        out_specs=[
            pl.BlockSpec((1, D), lambda i: (0, 0)),
            pl.BlockSpec((1, D), lambda i: (0, 0)),
        ],
        out_shape=[
            jax.ShapeDtypeStruct((1, D), jnp.float32),
            jax.ShapeDtypeStruct((1, D), jnp.float32),
        ],
    )(agg2, h, w_ih, w_hh, b_ih2, b_hh2, cls_w_pad, cls_b_pad)
    return probs


def kernel(x, edge_index, ggnn_weight, w_ih, w_hh, b_ih, b_hh, cls_w, cls_b):
    src = edge_index[0]
    dst = edge_index[1]
    pad_e = E_PAD - E
    src_p = jnp.concatenate([src, jnp.zeros((pad_e,), jnp.int32)])
    dst_p = jnp.concatenate([dst, jnp.full((pad_e,), N, jnp.int32)])
    packed = ((src_p << 16) | dst_p).reshape(NW, CPT, CHUNK)

    b_ih2 = b_ih.reshape(1, 3 * D)
    b_hh2 = b_hh.reshape(1, 3 * D)
    cls_w_pad = jnp.zeros((D, D), jnp.float32).at[:2].set(cls_w)
    cls_b_pad = jnp.full((1, D), -1e30, jnp.float32).at[0, :2].set(cls_b)

    h = x
    m = _tc_first(x, ggnn_weight[0])
    for i in range(NUM_STEPS):
        agg2 = _sc_edge_agg(m, packed)
        if i < NUM_STEPS - 1:
            h, m = _tc_step(agg2, h, w_ih, w_hh, b_ih2, b_hh2,
                            ggnn_weight[i + 1])
        else:
            probs = _tc_last(agg2, h, w_ih, w_hh, b_ih2, b_hh2,
                             cls_w_pad, cls_b_pad)
    return probs[:, :2]


# speedup vs baseline: 2.0095x; 2.0095x over previous
import functools

import jax
import jax.numpy as jnp
from jax import lax
from jax.experimental import pallas as pl
from jax.experimental.pallas import tpu as pltpu
from jax.experimental.pallas import tpu_sc as plsc

N = 10000
E = 320000
D = 128
NUM_STEPS = 4

NC = 2
NS = 16
NW = NC * NS
CHUNK = 80
CPT = 126
NBUF = 3
ZCH = 32
E_PAD = NW * CPT * CHUNK
NPAD = 10240
RPT = NPAD // NS

BN = 1000
GRID = N // BN


@functools.partial(
    pl.kernel,
    out_type=jax.ShapeDtypeStruct((NC, NPAD, D), jnp.float32),
    mesh=plsc.VectorSubcoreMesh(core_axis_name="c", subcore_axis_name="s"),
    scratch_types=[
        pltpu.VMEM((CPT, CHUNK), jnp.int32),
        pltpu.VMEM((NBUF, CHUNK), jnp.int32),
        pltpu.VMEM((NBUF, CHUNK), jnp.int32),
        [pltpu.VMEM((CHUNK, D), jnp.float32) for _ in range(NBUF)],
        pltpu.VMEM_SHARED((NPAD, D), jnp.float32),
        [pltpu.SemaphoreType.DMA for _ in range(NBUF)],
    ],
)
def _sc_edge_agg(m_hbm, pk_hbm, out_hbm, pk_v, sidx, didx, rows,
                 agg_sh, gsems):
    c = lax.axis_index("c")
    s = lax.axis_index("s")
    wid = c * NS + s

    zv = jnp.zeros((16,), jnp.float32)

    def zrow(i, carry):
        for k in range(D // 16):
            rows[0][i, pl.ds(k * 16, 16)] = zv
        return carry

    lax.fori_loop(0, ZCH, zrow, 0)

    def zslab(r, carry):
        pltpu.sync_copy(rows[0].at[pl.ds(0, ZCH)],
                        agg_sh.at[pl.ds(s * RPT + r * ZCH, ZCH)])
        return carry

    lax.fori_loop(0, RPT // ZCH, zslab, 0)

    pltpu.sync_copy(pk_hbm.at[wid], pk_v)

    plsc.subcore_barrier()

    def unpack(j, b):
        for k in range(CHUNK // 16):
            w = pk_v[j, pl.ds(k * 16, 16)]
            sidx[b, pl.ds(k * 16, 16)] = w >> 16
            didx[b, pl.ds(k * 16, 16)] = w & 0xFFFF

    def fire(b):
        pltpu.async_copy(m_hbm.at[sidx.at[b]], rows[b], gsems[b])

    def drain(b):
        pltpu.make_async_copy(m_hbm.at[sidx.at[b]], rows[b], gsems[b]).wait()

    def scat(b):
        pltpu.sync_copy(rows[b], agg_sh.at[didx.at[b]], add=True)

    for b in range(NBUF):
        unpack(b, b)
        fire(b)

    def body(u, carry):
        for b in range(NBUF):
            j = u * NBUF + b
            drain(b)
            scat(b)

            @pl.when(j + NBUF < CPT)
            def _():
                unpack(j + NBUF, b)
                fire(b)
        return carry

    lax.fori_loop(0, CPT // NBUF, body, 0)

    plsc.subcore_barrier()
    pltpu.sync_copy(agg_sh.at[pl.ds(s * RPT, RPT)],
                    out_hbm.at[c, pl.ds(s * RPT, RPT)])


_CONTRACT_LAST = (((1,), (1,)), ((), ()))


def _tc_first(x, w0):
    def body(x_ref, w_ref, m_ref):
        m_ref[...] = jnp.dot(x_ref[...], w_ref[...],
                             preferred_element_type=jnp.float32)

    return pl.pallas_call(
        body,
        grid=(GRID,),
        in_specs=[
            pl.BlockSpec((BN, D), lambda i: (i, 0)),
            pl.BlockSpec((D, D), lambda i: (0, 0)),
        ],
        out_specs=pl.BlockSpec((BN, D), lambda i: (i, 0)),
        out_shape=jax.ShapeDtypeStruct((N, D), jnp.float32),
    )(x, w0)


def _gru_block(agg_ref, h_ref, wih_ref, whh_ref, bih_ref, bhh_ref):
    agg = agg_ref[0] + agg_ref[1]
    h = h_ref[...]
    gi = lax.dot_general(agg, wih_ref[...], _CONTRACT_LAST,
                         preferred_element_type=jnp.float32) + bih_ref[...]
    gh = lax.dot_general(h, whh_ref[...], _CONTRACT_LAST,
                         preferred_element_type=jnp.float32) + bhh_ref[...]
    r = jax.nn.sigmoid(gi[:, 0:D] + gh[:, 0:D])
    z = jax.nn.sigmoid(gi[:, D:2 * D] + gh[:, D:2 * D])
    n = jnp.tanh(gi[:, 2 * D:3 * D] + r * gh[:, 2 * D:3 * D])
    return (1.0 - z) * n + z * h


def _tc_step(agg2, h, w_ih, w_hh, b_ih2, b_hh2, w_next):
    def body(agg_ref, h_ref, wih_ref, whh_ref, bih_ref, bhh_ref, wn_ref,
             hn_ref, mn_ref):
        h_new = _gru_block(agg_ref, h_ref, wih_ref, whh_ref, bih_ref, bhh_ref)
        hn_ref[...] = h_new
        mn_ref[...] = jnp.dot(h_new, wn_ref[...],
                              preferred_element_type=jnp.float32)

    return pl.pallas_call(
        body,
        grid=(GRID,),
        in_specs=[
            pl.BlockSpec((2, BN, D), lambda i: (0, i, 0)),
            pl.BlockSpec((BN, D), lambda i: (i, 0)),
            pl.BlockSpec((3 * D, D), lambda i: (0, 0)),
            pl.BlockSpec((3 * D, D), lambda i: (0, 0)),
            pl.BlockSpec((1, 3 * D), lambda i: (0, 0)),
            pl.BlockSpec((1, 3 * D), lambda i: (0, 0)),
            pl.BlockSpec((D, D), lambda i: (0, 0)),
        ],
        out_specs=[
            pl.BlockSpec((BN, D), lambda i: (i, 0)),
            pl.BlockSpec((BN, D), lambda i: (i, 0)),
        ],
        out_shape=[
            jax.ShapeDtypeStruct((N, D), jnp.float32),
            jax.ShapeDtypeStruct((N, D), jnp.float32),
        ],
    )(agg2, h, w_ih, w_hh, b_ih2, b_hh2, w_next)


def _tc_last(agg2, h, w_ih, w_hh, b_ih2, b_hh2, cls_w_pad, cls_b_pad):
    def body(agg_ref, h_ref, wih_ref, whh_ref, bih_ref, bhh_ref, cw_ref,
             cb_ref, pool_ref, probs_ref):
        i = pl.program_id(0)
        h_new = _gru_block(agg_ref, h_ref, wih_ref, whh_ref, bih_ref, bhh_ref)
        blk_max = jnp.max(jax.nn.relu(h_new), axis=0, keepdims=True)

        @pl.when(i == 0)
        def _():
            pool_ref[...] = blk_max

        @pl.when(i > 0)
        def _():
            pool_ref[...] = jnp.maximum(pool_ref[...], blk_max)

        @pl.when(i == GRID - 1)
        def _():
            logits = lax.dot_general(pool_ref[...], cw_ref[...],
                                     _CONTRACT_LAST,
                                     preferred_element_type=jnp.float32)
            logits = logits + cb_ref[...]
            e = jnp.exp(logits - jnp.max(logits, axis=1, keepdims=True))
            probs_ref[...] = e / jnp.sum(e, axis=1, keepdims=True)

    _, probs = pl.pallas_call(
        body,
        grid=(GRID,),
        in_specs=[
            pl.BlockSpec((2, BN, D), lambda i: (0, i, 0)),
            pl.BlockSpec((BN, D), lambda i: (i, 0)),
            pl.BlockSpec((3 * D, D), lambda i: (0, 0)),
            pl.BlockSpec((3 * D, D), lambda i: (0, 0)),
            pl.BlockSpec((1, 3 * D), lambda i: (0, 0)),
            pl.BlockSpec((1, 3 * D), lambda i: (0, 0)),
            pl.BlockSpec((D, D), lambda i: (0, 0)),
            pl.BlockSpec((1, D), lambda i: (0, 0)),
        ],
        out_specs=[
            pl.BlockSpec((1, D), lambda i: (0, 0)),
            pl.BlockSpec((1, D), lambda i: (0, 0)),
        ],
        out_shape=[
            jax.ShapeDtypeStruct((1, D), jnp.float32),
            jax.ShapeDtypeStruct((1, D), jnp.float32),
        ],
    )(agg2, h, w_ih, w_hh, b_ih2, b_hh2, cls_w_pad, cls_b_pad)
    return probs


def kernel(x, edge_index, ggnn_weight, w_ih, w_hh, b_ih, b_hh, cls_w, cls_b):
    src = edge_index[0]
    dst = edge_index[1]
    pad_e = E_PAD - E
    src_p = jnp.concatenate([src, jnp.zeros((pad_e,), jnp.int32)])
    dst_p = jnp.concatenate([dst, jnp.full((pad_e,), N, jnp.int32)])
    packed = ((src_p << 16) | dst_p).reshape(NW, CPT, CHUNK)

    b_ih2 = b_ih.reshape(1, 3 * D)
    b_hh2 = b_hh.reshape(1, 3 * D)
    cls_w_pad = jnp.zeros((D, D), jnp.float32).at[:2].set(cls_w)
    cls_b_pad = jnp.full((1, D), -1e30, jnp.float32).at[0, :2].set(cls_b)

    h = x
    m = _tc_first(x, ggnn_weight[0])
    for i in range(NUM_STEPS):
        agg2 = _sc_edge_agg(m, packed)
        if i < NUM_STEPS - 1:
            h, m = _tc_step(agg2, h, w_ih, w_hh, b_ih2, b_hh2,
                            ggnn_weight[i + 1])
        else:
            probs = _tc_last(agg2, h, w_ih, w_hh, b_ih2, b_hh2,
                             cls_w_pad, cls_b_pad)
    return probs[:, :2]
